# baseline (speedup 1.0000x reference)
import dataclasses
import functools

import jax
import jax.numpy as jnp
from jax import lax
from jax.experimental import pallas as pl
from jax.experimental.pallas import tpu as pltpu
from jax.experimental.pallas import tpu_sc as plsc

_LANES = 16
_NUM_WORKERS = 32
_WAVE = 128
_PANEL = 4096


def _tc_pack(table_t, eye):
    dim, buckets = table_t.shape
    n_blk = (buckets + 2 * _PANEL - 1) // (2 * _PANEL)

    max_panel = (buckets - 1) // _PANEL

    def body(xa_ref, xb_ref, eye_ref, out_ref):
        def xpose(x):
            return lax.dot_general(
                x[...].astype(jnp.bfloat16),
                eye_ref[...],
                dimension_numbers=(((0,), (0,)), ((), ())),
                preferred_element_type=jnp.float32,
            )

        out_ref[...] = jnp.concatenate([xpose(xa_ref), xpose(xb_ref)], axis=1)

    pack_params = pltpu.CompilerParams(
        dimension_semantics=("arbitrary",),
        fuse_transposed_lhs_in_matmul=True,
    )
    return pl.pallas_call(
        body,
        grid=(n_blk,),
        in_specs=[
            pl.BlockSpec((dim, _PANEL), lambda i: (jnp.int32(0), 2 * i)),
            pl.BlockSpec(
                (dim, _PANEL),
                lambda i: (jnp.int32(0), jnp.minimum(2 * i + 1, max_panel)),
            ),
            pl.BlockSpec((dim, dim), lambda i: (jnp.int32(0), jnp.int32(0))),
        ],
        out_specs=pl.BlockSpec((_PANEL, 2 * dim), lambda i: (i, jnp.int32(0))),
        out_shape=jax.ShapeDtypeStruct((n_blk * _PANEL, 2 * dim), jnp.float32),
        compiler_params=pack_params,
    )(table_t, table_t, eye)


def _sc_hash_gather(ids, tpad, seqlen, buckets):
    n_tok = ids.shape[0]
    chunk = n_tok // _NUM_WORKERS
    n_wave = chunk // _WAVE
    mesh = plsc.VectorSubcoreMesh(core_axis_name="c", subcore_axis_name="s")
    cparams = pltpu.CompilerParams(use_tc_tiling_on_sc=True)
    if "needs_layout_passes" in pltpu.CompilerParams.__dataclass_fields__:
        cparams = dataclasses.replace(cparams, needs_layout_passes=False)

    @functools.partial(
        pl.kernel,
        out_type=[
            jax.ShapeDtypeStruct((n_tok, 128), jnp.float32),
            jax.ShapeDtypeStruct((n_tok // _WAVE, _WAVE), jnp.int32),
        ],
        mesh=mesh,
        compiler_params=cparams,
        scratch_types=[
            pltpu.VMEM((_LANES + chunk,), jnp.int32),
            pltpu.VMEM((n_wave, _WAVE), jnp.int32),
            pltpu.VMEM((n_wave, _WAVE), jnp.int32),
            pltpu.VMEM((2, _WAVE, 128), jnp.float32),
            pltpu.SemaphoreType.DMA,
            pltpu.SemaphoreType.DMA,
        ],
    )
    def gather_kernel(
        ids_hbm, tp_hbm, emb_hbm, h_hbm, ids_pad, h_ref, r_ref, pad, gsem, wsem
    ):
        i32 = jnp.int32
        sub = lax.convert_element_type(lax.axis_index("s"), jnp.int32)
        core = lax.convert_element_type(lax.axis_index("c"), jnp.int32)
        wid = sub * i32(2) + core
        base = wid * i32(chunk)

        @pl.when(base % i32(seqlen) == i32(0))
        def _():
            ids_pad[pl.ds(0, _LANES)] = jnp.zeros((_LANES,), jnp.int32)
            pltpu.sync_copy(
                ids_hbm.at[pl.ds(base, chunk)], ids_pad.at[pl.ds(_LANES, chunk)]
            )

        @pl.when(base % i32(seqlen) != i32(0))
        def _():
            pltpu.sync_copy(
                ids_hbm.at[pl.ds(base - i32(_LANES), chunk + _LANES)], ids_pad
            )

        lane = lax.iota(jnp.int32, _LANES)

        @pl.loop(i32(0), i32(n_wave))
        def _(w):
            w = lax.convert_element_type(w, jnp.int32)
            for t in range(_WAVE // _LANES):
                off = w * i32(_WAVE) + i32(t * _LANES)
                cur = ids_pad[pl.ds(off + i32(_LANES), _LANES)]
                prev = plsc.load_gather(ids_pad, [lane + (off + i32(_LANES - 1))])
                h = (((prev * i32(92)) % i32(1000)) * i32(1000)
                     + prev * i32(821) + cur) % i32(buckets)
                h_ref[w, pl.ds(i32(t * _LANES), _LANES)] = h
                r_ref[w, pl.ds(i32(t * _LANES), _LANES)] = (
                    (h >> 13) * i32(_PANEL) + (h & i32(_PANEL - 1))
                )

        pltpu.sync_copy(h_ref, h_hbm.at[pl.ds(wid * i32(n_wave), n_wave)])

        def gather_start(w, buf):
            pltpu.make_async_copy(
                tp_hbm.at[r_ref.at[w]], pad.at[buf], gsem
            ).start()

        def gather_wait(w, buf):
            pltpu.make_async_copy(
                tp_hbm.at[r_ref.at[w]], pad.at[buf], gsem
            ).wait()

        def emb_start(w, buf):
            pltpu.make_async_copy(
                pad.at[buf],
                emb_hbm.at[pl.ds(base + w * i32(_WAVE), _WAVE)],
                wsem,
            ).start()

        def emb_wait(w, buf):
            pltpu.make_async_copy(
                pad.at[buf],
                emb_hbm.at[pl.ds(base + w * i32(_WAVE), _WAVE)],
                wsem,
            ).wait()

        gather_start(i32(0), i32(0))

        @pl.loop(i32(0), i32(n_wave - 1))
        def _(w):
            w = lax.convert_element_type(w, jnp.int32)
            buf = w % i32(2)
            gather_start(w + i32(1), i32(1) - buf)
            gather_wait(w, buf)
            emb_start(w, buf)

            @pl.when(w > i32(0))
            def _():
                emb_wait(w - i32(1), i32(1) - buf)

        last = i32(n_wave - 1)
        lbuf = last % i32(2)
        gather_wait(last, lbuf)
        emb_start(last, lbuf)
        emb_wait(last - i32(1), i32(1) - lbuf)
        emb_wait(last, lbuf)

    return gather_kernel(ids, tpad)


def _tc_project(emb, h, w2):
    n_tok = emb.shape[0]
    model_dim = w2.shape[1]
    blk = 512

    def body(emb_ref, h_ref, w2_ref, out_ref):
        half = (h_ref[...] >> jnp.int32(12)) & jnp.int32(1)
        col_half = lax.broadcasted_iota(jnp.int32, (blk, 128), 1) // jnp.int32(64)
        sel = jnp.where(col_half == half, emb_ref[...], jnp.float32(0.0))
        out_ref[...] = lax.dot_general(
            sel.astype(jnp.bfloat16),
            w2_ref[...],
            dimension_numbers=(((1,), (0,)), ((), ())),
            preferred_element_type=jnp.float32,
        )

    return pl.pallas_call(
        body,
        grid=(n_tok // blk,),
        in_specs=[
            pl.BlockSpec((blk, 128), lambda i: (i, jnp.int32(0))),
            pl.BlockSpec((blk, 1), lambda i: (i, jnp.int32(0))),
            pl.BlockSpec((128, model_dim), lambda i: (jnp.int32(0), jnp.int32(0))),
        ],
        out_specs=pl.BlockSpec((blk, model_dim), lambda i: (i, jnp.int32(0))),
        out_shape=jax.ShapeDtypeStruct((n_tok, model_dim), jnp.float32),
    )(emb, h, w2)


def kernel(input_ids, table, W):
    bsz, seqlen = input_ids.shape
    buckets, dim = table.shape
    ids = input_ids.reshape(-1).astype(jnp.int32)
    eye = jnp.eye(dim, dtype=jnp.bfloat16)
    tpad = _tc_pack(table.T, eye)
    emb, h = _sc_hash_gather(ids, tpad, seqlen, buckets)
    w2 = jnp.tile(W.T, (2, 1)).astype(jnp.bfloat16)
    out = _tc_project(emb, h.reshape(-1, 1), w2)
    return out.reshape(bsz, seqlen, W.shape[0])


# speedup vs baseline: 1.1121x; 1.1121x over previous
import dataclasses
import functools

import jax
import jax.numpy as jnp
from jax import lax
from jax.experimental import pallas as pl
from jax.experimental.pallas import tpu as pltpu
from jax.experimental.pallas import tpu_sc as plsc

_LANES = 16
_NUM_WORKERS = 32
_WAVE = 128
_PANEL = 8192


def _tc_pack(table_t, eye):
    dim, buckets = table_t.shape
    n_blk = (buckets + 2 * _PANEL - 1) // (2 * _PANEL)

    max_panel = (buckets - 1) // _PANEL

    def body(xa_ref, xb_ref, eye_ref, out_ref):
        def xpose(x):
            return lax.dot_general(
                x[...].astype(jnp.bfloat16),
                eye_ref[...],
                dimension_numbers=(((0,), (0,)), ((), ())),
                preferred_element_type=jnp.float32,
            )

        out_ref[...] = jnp.concatenate([xpose(xa_ref), xpose(xb_ref)], axis=1)

    pack_params = pltpu.CompilerParams(
        dimension_semantics=("arbitrary",),
        fuse_transposed_lhs_in_matmul=True,
    )
    return pl.pallas_call(
        body,
        grid=(n_blk,),
        in_specs=[
            pl.BlockSpec((dim, _PANEL), lambda i: (jnp.int32(0), 2 * i)),
            pl.BlockSpec(
                (dim, _PANEL),
                lambda i: (jnp.int32(0), jnp.minimum(2 * i + 1, max_panel)),
            ),
            pl.BlockSpec((dim, dim), lambda i: (jnp.int32(0), jnp.int32(0))),
        ],
        out_specs=pl.BlockSpec((_PANEL, 2 * dim), lambda i: (i, jnp.int32(0))),
        out_shape=jax.ShapeDtypeStruct((n_blk * _PANEL, 2 * dim), jnp.float32),
        compiler_params=pack_params,
    )(table_t, table_t, eye)


def _sc_hash_gather(ids, tpad, seqlen, buckets):
    n_tok = ids.shape[0]
    chunk = n_tok // _NUM_WORKERS
    n_wave = chunk // _WAVE
    mesh = plsc.VectorSubcoreMesh(core_axis_name="c", subcore_axis_name="s")
    cparams = pltpu.CompilerParams(use_tc_tiling_on_sc=True)
    if "needs_layout_passes" in pltpu.CompilerParams.__dataclass_fields__:
        cparams = dataclasses.replace(cparams, needs_layout_passes=False)

    @functools.partial(
        pl.kernel,
        out_type=[
            jax.ShapeDtypeStruct((n_tok, 128), jnp.float32),
            jax.ShapeDtypeStruct((n_tok // _WAVE, _WAVE), jnp.int32),
        ],
        mesh=mesh,
        compiler_params=cparams,
        scratch_types=[
            pltpu.VMEM((_LANES + chunk,), jnp.int32),
            pltpu.VMEM((n_wave, _WAVE), jnp.int32),
            pltpu.VMEM((n_wave, _WAVE), jnp.int32),
            pltpu.VMEM((2, _WAVE, 128), jnp.float32),
            pltpu.SemaphoreType.DMA,
            pltpu.SemaphoreType.DMA,
        ],
    )
    def gather_kernel(
        ids_hbm, tp_hbm, emb_hbm, h_hbm, ids_pad, h_ref, r_ref, pad, gsem, wsem
    ):
        i32 = jnp.int32
        sub = lax.convert_element_type(lax.axis_index("s"), jnp.int32)
        core = lax.convert_element_type(lax.axis_index("c"), jnp.int32)
        wid = sub * i32(2) + core
        base = wid * i32(chunk)

        @pl.when(base % i32(seqlen) == i32(0))
        def _():
            ids_pad[pl.ds(0, _LANES)] = jnp.zeros((_LANES,), jnp.int32)
            pltpu.sync_copy(
                ids_hbm.at[pl.ds(base, chunk)], ids_pad.at[pl.ds(_LANES, chunk)]
            )

        @pl.when(base % i32(seqlen) != i32(0))
        def _():
            pltpu.sync_copy(
                ids_hbm.at[pl.ds(base - i32(_LANES), chunk + _LANES)], ids_pad
            )

        lane = lax.iota(jnp.int32, _LANES)

        @pl.loop(i32(0), i32(n_wave))
        def _(w):
            w = lax.convert_element_type(w, jnp.int32)
            for t in range(_WAVE // _LANES):
                off = w * i32(_WAVE) + i32(t * _LANES)
                cur = ids_pad[pl.ds(off + i32(_LANES), _LANES)]
                prev = plsc.load_gather(ids_pad, [lane + (off + i32(_LANES - 1))])
                h = (((prev * i32(92)) % i32(1000)) * i32(1000)
                     + prev * i32(821) + cur) % i32(buckets)
                h_ref[w, pl.ds(i32(t * _LANES), _LANES)] = h
                r_ref[w, pl.ds(i32(t * _LANES), _LANES)] = (
                    (h >> i32(1 + _PANEL.bit_length() - 1))
                    * i32(_PANEL) + (h & i32(_PANEL - 1))
                )

        pltpu.sync_copy(h_ref, h_hbm.at[pl.ds(wid * i32(n_wave), n_wave)])

        def gather_start(w, buf):
            pltpu.make_async_copy(
                tp_hbm.at[r_ref.at[w]], pad.at[buf], gsem
            ).start()

        def gather_wait(w, buf):
            pltpu.make_async_copy(
                tp_hbm.at[r_ref.at[w]], pad.at[buf], gsem
            ).wait()

        def emb_start(w, buf):
            pltpu.make_async_copy(
                pad.at[buf],
                emb_hbm.at[pl.ds(base + w * i32(_WAVE), _WAVE)],
                wsem,
            ).start()

        def emb_wait(w, buf):
            pltpu.make_async_copy(
                pad.at[buf],
                emb_hbm.at[pl.ds(base + w * i32(_WAVE), _WAVE)],
                wsem,
            ).wait()

        gather_start(i32(0), i32(0))

        @pl.loop(i32(0), i32(n_wave - 1))
        def _(w):
            w = lax.convert_element_type(w, jnp.int32)
            buf = w % i32(2)
            gather_start(w + i32(1), i32(1) - buf)
            gather_wait(w, buf)
            emb_start(w, buf)

            @pl.when(w > i32(0))
            def _():
                emb_wait(w - i32(1), i32(1) - buf)

        last = i32(n_wave - 1)
        lbuf = last % i32(2)
        gather_wait(last, lbuf)
        emb_start(last, lbuf)
        emb_wait(last - i32(1), i32(1) - lbuf)
        emb_wait(last, lbuf)

    return gather_kernel(ids, tpad)


def _tc_project(emb, h, w2):
    n_tok = emb.shape[0]
    model_dim = w2.shape[1]
    blk = 512

    def body(emb_ref, h_ref, w2_ref, out_ref):
        half = (h_ref[...] >> jnp.int32(_PANEL.bit_length() - 1)) & jnp.int32(1)
        col_half = lax.broadcasted_iota(jnp.int32, (blk, 128), 1) // jnp.int32(64)
        sel = jnp.where(col_half == half, emb_ref[...], jnp.float32(0.0))
        out_ref[...] = lax.dot_general(
            sel.astype(jnp.bfloat16),
            w2_ref[...],
            dimension_numbers=(((1,), (0,)), ((), ())),
            preferred_element_type=jnp.float32,
        )

    return pl.pallas_call(
        body,
        grid=(n_tok // blk,),
        in_specs=[
            pl.BlockSpec((blk, 128), lambda i: (i, jnp.int32(0))),
            pl.BlockSpec((blk, 1), lambda i: (i, jnp.int32(0))),
            pl.BlockSpec((128, model_dim), lambda i: (jnp.int32(0), jnp.int32(0))),
        ],
        out_specs=pl.BlockSpec((blk, model_dim), lambda i: (i, jnp.int32(0))),
        out_shape=jax.ShapeDtypeStruct((n_tok, model_dim), jnp.float32),
    )(emb, h, w2)


def kernel(input_ids, table, W):
    bsz, seqlen = input_ids.shape
    buckets, dim = table.shape
    ids = input_ids.reshape(-1).astype(jnp.int32)
    eye = jnp.eye(dim, dtype=jnp.bfloat16)
    tpad = _tc_pack(table.T, eye)
    emb, h = _sc_hash_gather(ids, tpad, seqlen, buckets)
    w2 = jnp.tile(W.T, (2, 1)).astype(jnp.bfloat16)
    out = _tc_project(emb, h.reshape(-1, 1), w2)
    return out.reshape(bsz, seqlen, W.shape[0])


# speedup vs baseline: 1.1827x; 1.0635x over previous
import dataclasses
import functools

import jax
import jax.numpy as jnp
from jax import lax
from jax.experimental import pallas as pl
from jax.experimental.pallas import tpu as pltpu
from jax.experimental.pallas import tpu_sc as plsc

_LANES = 16
_NUM_WORKERS = 32
_WAVE = 128
_PANEL = 16384


def _tc_pack(table_t, eye):
    dim, buckets = table_t.shape
    n_blk = (buckets + 2 * _PANEL - 1) // (2 * _PANEL)

    max_panel = (buckets - 1) // _PANEL

    def body(xa_ref, xb_ref, eye_ref, out_ref):
        def xpose(x):
            return lax.dot_general(
                x[...].astype(jnp.bfloat16),
                eye_ref[...],
                dimension_numbers=(((0,), (0,)), ((), ())),
                preferred_element_type=jnp.float32,
            )

        out_ref[...] = jnp.concatenate([xpose(xa_ref), xpose(xb_ref)], axis=1)

    pack_params = pltpu.CompilerParams(
        dimension_semantics=("arbitrary",),
        fuse_transposed_lhs_in_matmul=True,
    )
    return pl.pallas_call(
        body,
        grid=(n_blk,),
        in_specs=[
            pl.BlockSpec((dim, _PANEL), lambda i: (jnp.int32(0), 2 * i)),
            pl.BlockSpec(
                (dim, _PANEL),
                lambda i: (jnp.int32(0), jnp.minimum(2 * i + 1, max_panel)),
            ),
            pl.BlockSpec((dim, dim), lambda i: (jnp.int32(0), jnp.int32(0))),
        ],
        out_specs=pl.BlockSpec((_PANEL, 2 * dim), lambda i: (i, jnp.int32(0))),
        out_shape=jax.ShapeDtypeStruct((n_blk * _PANEL, 2 * dim), jnp.float32),
        compiler_params=pack_params,
    )(table_t, table_t, eye)


def _sc_hash_gather(ids, tpad, seqlen, buckets):
    n_tok = ids.shape[0]
    chunk = n_tok // _NUM_WORKERS
    n_wave = chunk // _WAVE
    mesh = plsc.VectorSubcoreMesh(core_axis_name="c", subcore_axis_name="s")
    cparams = pltpu.CompilerParams(use_tc_tiling_on_sc=True)
    if "needs_layout_passes" in pltpu.CompilerParams.__dataclass_fields__:
        cparams = dataclasses.replace(cparams, needs_layout_passes=False)

    @functools.partial(
        pl.kernel,
        out_type=[
            jax.ShapeDtypeStruct((n_tok, 128), jnp.float32),
            jax.ShapeDtypeStruct((n_tok // _WAVE, _WAVE), jnp.int32),
        ],
        mesh=mesh,
        compiler_params=cparams,
        scratch_types=[
            pltpu.VMEM((_LANES + chunk,), jnp.int32),
            pltpu.VMEM((n_wave, _WAVE), jnp.int32),
            pltpu.VMEM((n_wave, _WAVE), jnp.int32),
            pltpu.VMEM((2, _WAVE, 128), jnp.float32),
            pltpu.SemaphoreType.DMA,
            pltpu.SemaphoreType.DMA,
        ],
    )
    def gather_kernel(
        ids_hbm, tp_hbm, emb_hbm, h_hbm, ids_pad, h_ref, r_ref, pad, gsem, wsem
    ):
        i32 = jnp.int32
        sub = lax.convert_element_type(lax.axis_index("s"), jnp.int32)
        core = lax.convert_element_type(lax.axis_index("c"), jnp.int32)
        wid = sub * i32(2) + core
        base = wid * i32(chunk)

        @pl.when(base % i32(seqlen) == i32(0))
        def _():
            ids_pad[pl.ds(0, _LANES)] = jnp.zeros((_LANES,), jnp.int32)
            pltpu.sync_copy(
                ids_hbm.at[pl.ds(base, chunk)], ids_pad.at[pl.ds(_LANES, chunk)]
            )

        @pl.when(base % i32(seqlen) != i32(0))
        def _():
            pltpu.sync_copy(
                ids_hbm.at[pl.ds(base - i32(_LANES), chunk + _LANES)], ids_pad
            )

        lane = lax.iota(jnp.int32, _LANES)

        @pl.loop(i32(0), i32(n_wave))
        def _(w):
            w = lax.convert_element_type(w, jnp.int32)
            for t in range(_WAVE // _LANES):
                off = w * i32(_WAVE) + i32(t * _LANES)
                cur = ids_pad[pl.ds(off + i32(_LANES), _LANES)]
                prev = plsc.load_gather(ids_pad, [lane + (off + i32(_LANES - 1))])
                h = (((prev * i32(92)) % i32(1000)) * i32(1000)
                     + prev * i32(821) + cur) % i32(buckets)
                h_ref[w, pl.ds(i32(t * _LANES), _LANES)] = h
                r_ref[w, pl.ds(i32(t * _LANES), _LANES)] = (
                    (h >> i32(1 + _PANEL.bit_length() - 1))
                    * i32(_PANEL) + (h & i32(_PANEL - 1))
                )

        pltpu.sync_copy(h_ref, h_hbm.at[pl.ds(wid * i32(n_wave), n_wave)])

        def gather_start(w, buf):
            pltpu.make_async_copy(
                tp_hbm.at[r_ref.at[w]], pad.at[buf], gsem
            ).start()

        def gather_wait(w, buf):
            pltpu.make_async_copy(
                tp_hbm.at[r_ref.at[w]], pad.at[buf], gsem
            ).wait()

        def emb_start(w, buf):
            pltpu.make_async_copy(
                pad.at[buf],
                emb_hbm.at[pl.ds(base + w * i32(_WAVE), _WAVE)],
                wsem,
            ).start()

        def emb_wait(w, buf):
            pltpu.make_async_copy(
                pad.at[buf],
                emb_hbm.at[pl.ds(base + w * i32(_WAVE), _WAVE)],
                wsem,
            ).wait()

        gather_start(i32(0), i32(0))

        @pl.loop(i32(0), i32(n_wave - 1))
        def _(w):
            w = lax.convert_element_type(w, jnp.int32)
            buf = w % i32(2)
            gather_start(w + i32(1), i32(1) - buf)
            gather_wait(w, buf)
            emb_start(w, buf)

            @pl.when(w > i32(0))
            def _():
                emb_wait(w - i32(1), i32(1) - buf)

        last = i32(n_wave - 1)
        lbuf = last % i32(2)
        gather_wait(last, lbuf)
        emb_start(last, lbuf)
        emb_wait(last - i32(1), i32(1) - lbuf)
        emb_wait(last, lbuf)

    return gather_kernel(ids, tpad)


def _tc_project(emb, h, w2):
    n_tok = emb.shape[0]
    model_dim = w2.shape[1]
    blk = 512

    def body(emb_ref, h_ref, w2_ref, out_ref):
        half = (h_ref[...] >> jnp.int32(_PANEL.bit_length() - 1)) & jnp.int32(1)
        col_half = lax.broadcasted_iota(jnp.int32, (blk, 128), 1) // jnp.int32(64)
        sel = jnp.where(col_half == half, emb_ref[...], jnp.float32(0.0))
        out_ref[...] = lax.dot_general(
            sel.astype(jnp.bfloat16),
            w2_ref[...],
            dimension_numbers=(((1,), (0,)), ((), ())),
            preferred_element_type=jnp.float32,
        )

    return pl.pallas_call(
        body,
        grid=(n_tok // blk,),
        in_specs=[
            pl.BlockSpec((blk, 128), lambda i: (i, jnp.int32(0))),
            pl.BlockSpec((blk, 1), lambda i: (i, jnp.int32(0))),
            pl.BlockSpec((128, model_dim), lambda i: (jnp.int32(0), jnp.int32(0))),
        ],
        out_specs=pl.BlockSpec((blk, model_dim), lambda i: (i, jnp.int32(0))),
        out_shape=jax.ShapeDtypeStruct((n_tok, model_dim), jnp.float32),
    )(emb, h, w2)


def kernel(input_ids, table, W):
    bsz, seqlen = input_ids.shape
    buckets, dim = table.shape
    ids = input_ids.reshape(-1).astype(jnp.int32)
    eye = jnp.eye(dim, dtype=jnp.bfloat16)
    tpad = _tc_pack(table.T, eye)
    emb, h = _sc_hash_gather(ids, tpad, seqlen, buckets)
    w2 = jnp.tile(W.T, (2, 1)).astype(jnp.bfloat16)
    out = _tc_project(emb, h.reshape(-1, 1), w2)
    return out.reshape(bsz, seqlen, W.shape[0])
